# Initial kernel scaffold; baseline (speedup 1.0000x reference)
import jax
import jax.numpy as jnp
from jax.experimental import pallas as pl


def kernel(x, edge_index, W1, b1, W2, b2, W3, b3, Wl, bl):
    raise NotImplementedError("write your pallas kernel here")


# speedup vs baseline: 5.7676x; 5.7676x over previous
import functools

import jax
import jax.numpy as jnp
from jax import lax
from jax.experimental import pallas as pl
from jax.experimental.pallas import tpu as pltpu
from jax.experimental.pallas import tpu_sc as plsc

N = 10000
D = 128
C = 21
E = 320000

NC = 2
NS = 16
NW = NC * NS
K = 128
CHUNKS = 80
EPW = K * CHUNKS
E_PAD = EPW * NW
NACC = 10112
RPT = NACC // NS

_sc_mesh = plsc.VectorSubcoreMesh(core_axis_name="c", subcore_axis_name="s")


def _deg_body(dst_hbm, ones_hbm, zacc_hbm, out_hbm, deg_sh, idx_v, ones_v, sem):
    cid = lax.axis_index("c")
    sid = lax.axis_index("s")
    wid = cid * NS + sid
    pltpu.sync_copy(zacc_hbm.at[pl.ds(sid * RPT, RPT)],
                    deg_sh.at[pl.ds(sid * RPT, RPT)])
    pltpu.sync_copy(ones_hbm, ones_v)
    plsc.subcore_barrier()

    def body(c, carry):
        off = pl.multiple_of(wid * EPW + c * K, K)
        pltpu.sync_copy(dst_hbm.at[pl.ds(off, K)], idx_v)
        pltpu.sync_copy(ones_v, deg_sh.at[idx_v], add=True)
        return carry

    lax.fori_loop(0, CHUNKS, body, 0)
    plsc.subcore_barrier()
    pltpu.sync_copy(deg_sh.at[pl.ds(sid * RPT, RPT)],
                    out_hbm.at[cid, pl.ds(sid * RPT, RPT)])


_deg_pass = functools.partial(
    pl.kernel,
    out_type=jax.ShapeDtypeStruct((NC, NACC, D), jnp.float32),
    mesh=_sc_mesh,
    scratch_types=[
        pltpu.VMEM_SHARED((NACC, D), jnp.float32),
        pltpu.VMEM((K,), jnp.int32),
        pltpu.VMEM((K, D), jnp.float32),
        pltpu.SemaphoreType.DMA,
    ],
)(_deg_body)


def _scat_body(g_hbm, src_hbm, dst_hbm, zacc_hbm, out_hbm,
               acc_sh, sidx_v, didx_v, rows_v, sem):
    cid = lax.axis_index("c")
    sid = lax.axis_index("s")
    wid = cid * NS + sid
    pltpu.sync_copy(zacc_hbm.at[pl.ds(sid * RPT, RPT)],
                    acc_sh.at[pl.ds(sid * RPT, RPT)])
    plsc.subcore_barrier()

    def body(c, carry):
        off = pl.multiple_of(wid * EPW + c * K, K)
        pltpu.sync_copy(src_hbm.at[pl.ds(off, K)], sidx_v)
        pltpu.sync_copy(dst_hbm.at[pl.ds(off, K)], didx_v)
        pltpu.async_copy(g_hbm.at[sidx_v], rows_v, sem).wait()
        pltpu.sync_copy(rows_v, acc_sh.at[didx_v], add=True)
        return carry

    lax.fori_loop(0, CHUNKS, body, 0)
    plsc.subcore_barrier()
    pltpu.sync_copy(acc_sh.at[pl.ds(sid * RPT, RPT)],
                    out_hbm.at[cid, pl.ds(sid * RPT, RPT)])


_scat_pass = functools.partial(
    pl.kernel,
    out_type=jax.ShapeDtypeStruct((NC, NACC, D), jnp.float32),
    mesh=_sc_mesh,
    scratch_types=[
        pltpu.VMEM_SHARED((NACC, D), jnp.float32),
        pltpu.VMEM((K,), jnp.int32),
        pltpu.VMEM((K,), jnp.int32),
        pltpu.VMEM((K, D), jnp.float32),
        pltpu.SemaphoreType.DMA,
    ],
)(_scat_body)


BR = 1264
_TC_GRID = NACC // BR


def _g1_body(x_ref, degp_ref, w_ref, g_ref, dinv_ref):
    deg = jnp.sum(degp_ref[0] + degp_ref[1], axis=-1) * (1.0 / D) + 1.0
    dinv = (1.0 / jnp.sqrt(deg))[:, None]
    dinv_ref[:, :] = jnp.broadcast_to(dinv, (BR, D))
    g_ref[:, :] = jnp.dot(x_ref[:, :], w_ref[:, :],
                          preferred_element_type=jnp.float32) * dinv


def _mid_body(acc_ref, g_ref, dinv_ref, b_ref, w_ref, out_ref):
    dinv = dinv_ref[:, :]
    v = (acc_ref[0] + acc_ref[1] + g_ref[:, :]) * dinv + b_ref[:]
    xn = jnp.maximum(v, 0.01 * v)
    out_ref[:, :] = jnp.dot(xn, w_ref[:, :],
                            preferred_element_type=jnp.float32) * dinv


def _final_body(acc_ref, g_ref, dinv_ref, b_ref, wl_ref, bl_ref, out_ref):
    dinv = dinv_ref[:, :]
    v = (acc_ref[0] + acc_ref[1] + g_ref[:, :]) * dinv + b_ref[:]
    xn = jnp.maximum(v, 0.01 * v)
    out_ref[:, :] = jnp.dot(xn, wl_ref[:, :],
                            preferred_element_type=jnp.float32) + bl_ref[:]


_row_spec = pl.BlockSpec((BR, D), lambda i: (i, 0))
_acc_spec = pl.BlockSpec((NC, BR, D), lambda i: (0, i, 0))
_w_spec = pl.BlockSpec((D, D), lambda i: (0, 0))
_b_spec = pl.BlockSpec((D,), lambda i: (0,))


def _tc_g1(x_pad, degp, w1):
    return pl.pallas_call(
        _g1_body,
        out_shape=[jax.ShapeDtypeStruct((NACC, D), jnp.float32),
                   jax.ShapeDtypeStruct((NACC, D), jnp.float32)],
        grid=(_TC_GRID,),
        in_specs=[_row_spec, _acc_spec, _w_spec],
        out_specs=[_row_spec, _row_spec],
    )(x_pad, degp, w1)


def _tc_mid(acc, g, dinv, b, w):
    return pl.pallas_call(
        _mid_body,
        out_shape=jax.ShapeDtypeStruct((NACC, D), jnp.float32),
        grid=(_TC_GRID,),
        in_specs=[_acc_spec, _row_spec, _row_spec, _b_spec, _w_spec],
        out_specs=_row_spec,
    )(acc, g, dinv, b, w)


def _tc_final(acc, g, dinv, b, wl, bl):
    return pl.pallas_call(
        _final_body,
        out_shape=jax.ShapeDtypeStruct((NACC, C), jnp.float32),
        grid=(_TC_GRID,),
        in_specs=[_acc_spec, _row_spec, _row_spec, _b_spec,
                  pl.BlockSpec((D, C), lambda i: (0, 0)),
                  pl.BlockSpec((C,), lambda i: (0,))],
        out_specs=pl.BlockSpec((BR, C), lambda i: (i, 0)),
    )(acc, g, dinv, b, wl, bl)


def kernel(x, edge_index, W1, b1, W2, b2, W3, b3, Wl, bl):
    ei = edge_index.astype(jnp.int32)
    pad = jnp.full((E_PAD - E,), N, dtype=jnp.int32)
    src = jnp.concatenate([ei[0], pad])
    dst = jnp.concatenate([ei[1], pad])
    x_pad = jnp.pad(x, ((0, NACC - N), (0, 0)))

    ones_rows = jnp.ones((K, D), jnp.float32)
    zacc = jnp.zeros((NACC, D), jnp.float32)

    degp = _deg_pass(dst, ones_rows, zacc)

    g1, dinv = _tc_g1(x_pad, degp, W1)
    acc1 = _scat_pass(g1, src, dst, zacc)
    g2 = _tc_mid(acc1, g1, dinv, b1, W2)
    acc2 = _scat_pass(g2, src, dst, zacc)
    g3 = _tc_mid(acc2, g2, dinv, b2, W3)
    acc3 = _scat_pass(g3, src, dst, zacc)
    out = _tc_final(acc3, g3, dinv, b3, Wl, bl)
    return out[:N]


# speedup vs baseline: 7.3215x; 1.2694x over previous
import functools

import jax
import jax.numpy as jnp
from jax import lax
from jax.experimental import pallas as pl
from jax.experimental.pallas import tpu as pltpu
from jax.experimental.pallas import tpu_sc as plsc

N = 10000
D = 128
C = 21
E = 320000

NC = 2
NS = 16
NW = NC * NS
K = 128
CH0 = 80
CH1 = 80
CH_MAX = max(CH0, CH1)
TOT_CHUNKS = NS * (CH0 + CH1)
E_PAD = K * TOT_CHUNKS
IDX_ROWS = TOT_CHUNKS + CH_MAX
NACC = 10112
RPT = NACC // NS

_sc_mesh = plsc.VectorSubcoreMesh(core_axis_name="c", subcore_axis_name="s")


def _chunk_base(cid, sid):
    base = jnp.where(cid == 0, sid * CH0, NS * CH0 + sid * CH1)
    cnt = jnp.where(cid == 0, CH0, CH1)
    return base, cnt


def _deg_body(dst_hbm, ones_hbm, zacc_hbm, out_hbm, deg_sh, didx_all, ones_v, sem):
    cid = lax.axis_index("c")
    sid = lax.axis_index("s")
    base, cnt = _chunk_base(cid, sid)
    pltpu.sync_copy(zacc_hbm.at[pl.ds(sid * RPT, RPT)],
                    deg_sh.at[pl.ds(sid * RPT, RPT)])
    pltpu.sync_copy(dst_hbm.at[pl.ds(base, CH_MAX)], didx_all)
    pltpu.sync_copy(ones_hbm, ones_v)
    plsc.subcore_barrier()

    def body(c, carry):
        @pl.when(c < cnt)
        def _():
            pltpu.sync_copy(ones_v, deg_sh.at[didx_all.at[c]], add=True)
        return carry

    lax.fori_loop(0, CH_MAX, body, 0)
    plsc.subcore_barrier()
    pltpu.sync_copy(deg_sh.at[pl.ds(sid * RPT, RPT)],
                    out_hbm.at[cid, pl.ds(sid * RPT, RPT)])


_deg_pass = functools.partial(
    pl.kernel,
    out_type=jax.ShapeDtypeStruct((NC, NACC, D), jnp.float32),
    mesh=_sc_mesh,
    scratch_types=[
        pltpu.VMEM_SHARED((NACC, D), jnp.float32),
        pltpu.VMEM((CH_MAX, K), jnp.int32),
        pltpu.VMEM((K, D), jnp.float32),
        pltpu.SemaphoreType.DMA,
    ],
)(_deg_body)


def _scat_body(g_hbm, idx_hbm, zacc_hbm, out_hbm,
               acc_sh, i0, i1, i2, i3, rows0, rows1,
               si0, si1, si2, si3, sg0, sg1):
    cid = lax.axis_index("c")
    sid = lax.axis_index("s")
    base, cnt = _chunk_base(cid, sid)
    ibuf = (i0, i1, i2, i3)
    isem = (si0, si1, si2, si3)
    rows = (rows0, rows1)
    gsem = (sg0, sg1)
    pltpu.sync_copy(zacc_hbm.at[pl.ds(sid * RPT, RPT)],
                    acc_sh.at[pl.ds(sid * RPT, RPT)])
    plsc.subcore_barrier()

    pltpu.async_copy(idx_hbm.at[base], ibuf[0], isem[0])
    pltpu.async_copy(idx_hbm.at[base + 1], ibuf[1], isem[1])
    pltpu.make_async_copy(idx_hbm.at[base], ibuf[0], isem[0]).wait()
    pltpu.async_copy(g_hbm.at[ibuf[0].at[0]], rows[0], gsem[0])

    def outer(c4, carry):
        for u in range(4):
            c = c4 * 4 + u
            b2 = u % 2
            bi = u

            @pl.when(c + 2 < cnt)
            def _():
                pltpu.async_copy(idx_hbm.at[base + c + 2],
                                 ibuf[(bi + 2) % 4], isem[(bi + 2) % 4])

            @pl.when(c + 1 < cnt)
            def _():
                pltpu.make_async_copy(idx_hbm.at[base],
                                      ibuf[(bi + 1) % 4],
                                      isem[(bi + 1) % 4]).wait()
                pltpu.async_copy(g_hbm.at[ibuf[(bi + 1) % 4].at[0]],
                                 rows[1 - b2], gsem[1 - b2])

            @pl.when(c < cnt)
            def _():
                pltpu.make_async_copy(g_hbm.at[pl.ds(0, K)],
                                      rows[b2], gsem[b2]).wait()
                pltpu.sync_copy(rows[b2], acc_sh.at[ibuf[bi].at[1]], add=True)
        return carry

    lax.fori_loop(0, CH_MAX // 4, outer, 0)
    plsc.subcore_barrier()
    pltpu.sync_copy(acc_sh.at[pl.ds(sid * RPT, RPT)],
                    out_hbm.at[cid, pl.ds(sid * RPT, RPT)])


_scat_pass = functools.partial(
    pl.kernel,
    out_type=jax.ShapeDtypeStruct((NC, NACC, D), jnp.float32),
    mesh=_sc_mesh,
    scratch_types=[
        pltpu.VMEM_SHARED((NACC, D), jnp.float32),
        pltpu.VMEM((2, K), jnp.int32),
        pltpu.VMEM((2, K), jnp.int32),
        pltpu.VMEM((2, K), jnp.int32),
        pltpu.VMEM((2, K), jnp.int32),
        pltpu.VMEM((K, D), jnp.float32),
        pltpu.VMEM((K, D), jnp.float32),
        pltpu.SemaphoreType.DMA,
        pltpu.SemaphoreType.DMA,
        pltpu.SemaphoreType.DMA,
        pltpu.SemaphoreType.DMA,
        pltpu.SemaphoreType.DMA,
        pltpu.SemaphoreType.DMA,
    ],
)(_scat_body)


BR = 1264
_TC_GRID = NACC // BR


def _g1_body(x_ref, degp_ref, w_ref, g_ref, dinv_ref):
    deg = jnp.sum(degp_ref[0] + degp_ref[1], axis=-1) * (1.0 / D) + 1.0
    dinv = (1.0 / jnp.sqrt(deg))[:, None]
    dinv_ref[:, :] = jnp.broadcast_to(dinv, (BR, D))
    g_ref[:, :] = jnp.dot(x_ref[:, :], w_ref[:, :],
                          preferred_element_type=jnp.float32) * dinv


def _mid_body(acc_ref, g_ref, dinv_ref, b_ref, w_ref, out_ref):
    dinv = dinv_ref[:, :]
    v = (acc_ref[0] + acc_ref[1] + g_ref[:, :]) * dinv + b_ref[:]
    xn = jnp.maximum(v, 0.01 * v)
    out_ref[:, :] = jnp.dot(xn, w_ref[:, :],
                            preferred_element_type=jnp.float32) * dinv


def _final_body(acc_ref, g_ref, dinv_ref, b_ref, wl_ref, bl_ref, out_ref):
    dinv = dinv_ref[:, :]
    v = (acc_ref[0] + acc_ref[1] + g_ref[:, :]) * dinv + b_ref[:]
    xn = jnp.maximum(v, 0.01 * v)
    out_ref[:, :] = jnp.dot(xn, wl_ref[:, :],
                            preferred_element_type=jnp.float32) + bl_ref[:]


_row_spec = pl.BlockSpec((BR, D), lambda i: (i, 0))
_acc_spec = pl.BlockSpec((NC, BR, D), lambda i: (0, i, 0))
_w_spec = pl.BlockSpec((D, D), lambda i: (0, 0))
_b_spec = pl.BlockSpec((D,), lambda i: (0,))


def _tc_g1(x_pad, degp, w1):
    return pl.pallas_call(
        _g1_body,
        out_shape=[jax.ShapeDtypeStruct((NACC, D), jnp.float32),
                   jax.ShapeDtypeStruct((NACC, D), jnp.float32)],
        grid=(_TC_GRID,),
        in_specs=[_row_spec, _acc_spec, _w_spec],
        out_specs=[_row_spec, _row_spec],
    )(x_pad, degp, w1)


def _tc_mid(acc, g, dinv, b, w):
    return pl.pallas_call(
        _mid_body,
        out_shape=jax.ShapeDtypeStruct((NACC, D), jnp.float32),
        grid=(_TC_GRID,),
        in_specs=[_acc_spec, _row_spec, _row_spec, _b_spec, _w_spec],
        out_specs=_row_spec,
    )(acc, g, dinv, b, w)


def _tc_final(acc, g, dinv, b, wl, bl):
    return pl.pallas_call(
        _final_body,
        out_shape=jax.ShapeDtypeStruct((NACC, C), jnp.float32),
        grid=(_TC_GRID,),
        in_specs=[_acc_spec, _row_spec, _row_spec, _b_spec,
                  pl.BlockSpec((D, C), lambda i: (0, 0)),
                  pl.BlockSpec((C,), lambda i: (0,))],
        out_specs=pl.BlockSpec((BR, C), lambda i: (i, 0)),
    )(acc, g, dinv, b, wl, bl)


def kernel(x, edge_index, W1, b1, W2, b2, W3, b3, Wl, bl):
    ei = edge_index.astype(jnp.int32)
    pad = jnp.full((IDX_ROWS * K - E,), N, dtype=jnp.int32)
    src = jnp.concatenate([ei[0], pad]).reshape(IDX_ROWS, K)
    dst = jnp.concatenate([ei[1], pad]).reshape(IDX_ROWS, K)
    idx2 = jnp.stack([src, dst], axis=1)
    x_pad = jnp.pad(x, ((0, NACC - N), (0, 0)))

    ones_rows = jnp.ones((K, D), jnp.float32)
    zacc = jnp.zeros((NACC, D), jnp.float32)

    degp = _deg_pass(dst, ones_rows, zacc)

    g1, dinv = _tc_g1(x_pad, degp, W1)
    acc1 = _scat_pass(g1, idx2, zacc)
    g2 = _tc_mid(acc1, g1, dinv, b1, W2)
    acc2 = _scat_pass(g2, idx2, zacc)
    g3 = _tc_mid(acc2, g2, dinv, b2, W3)
    acc3 = _scat_pass(g3, idx2, zacc)
    out = _tc_final(acc3, g3, dinv, b3, Wl, bl)
    return out[:N]


# speedup vs baseline: 7.4636x; 1.0194x over previous
import functools

import jax
import jax.numpy as jnp
from jax import lax
from jax.experimental import pallas as pl
from jax.experimental.pallas import tpu as pltpu
from jax.experimental.pallas import tpu_sc as plsc

N = 10000
D = 128
C = 21
E = 320000

NC = 2
NS = 16
NW = NC * NS
K = 128
CH0 = 32
CH1 = 128
CHD = 80
CH_MAX = max(CH0, CH1)
TOT_CHUNKS = NS * (CH0 + CH1)
E_PAD = K * TOT_CHUNKS
IDX_ROWS = TOT_CHUNKS + CH_MAX
NACC = 10112
RPT = NACC // NS

_sc_mesh = plsc.VectorSubcoreMesh(core_axis_name="c", subcore_axis_name="s")


def _chunk_base(cid, sid):
    base = jnp.where(cid == 0, sid * CH0, NS * CH0 + sid * CH1)
    cnt = jnp.where(cid == 0, CH0, CH1)
    return base, cnt


def _deg_body(dst_hbm, ones_hbm, zacc_hbm, out_hbm, deg_sh, didx_all, ones_v, sem):
    cid = lax.axis_index("c")
    sid = lax.axis_index("s")
    base = (cid * NS + sid) * CHD
    cnt = CHD
    pltpu.sync_copy(zacc_hbm.at[pl.ds(sid * RPT, RPT)],
                    deg_sh.at[pl.ds(sid * RPT, RPT)])
    pltpu.sync_copy(dst_hbm.at[pl.ds(base, CHD)], didx_all)
    pltpu.sync_copy(ones_hbm, ones_v)
    plsc.subcore_barrier()

    def body(c, carry):
        @pl.when(c < cnt)
        def _():
            pltpu.sync_copy(ones_v, deg_sh.at[didx_all.at[c]], add=True)
        return carry

    lax.fori_loop(0, CHD, body, 0)
    plsc.subcore_barrier()
    pltpu.sync_copy(deg_sh.at[pl.ds(sid * RPT, RPT)],
                    out_hbm.at[cid, pl.ds(sid * RPT, RPT)])


_deg_pass = functools.partial(
    pl.kernel,
    out_type=jax.ShapeDtypeStruct((NC, NACC, D), jnp.float32),
    mesh=_sc_mesh,
    scratch_types=[
        pltpu.VMEM_SHARED((NACC, D), jnp.float32),
        pltpu.VMEM((CHD, K), jnp.int32),
        pltpu.VMEM((K, D), jnp.float32),
        pltpu.SemaphoreType.DMA,
    ],
)(_deg_body)


def _scat_body(g_hbm, idx_hbm, zacc_hbm, out_hbm,
               acc_sh, i0, i1, i2, i3, rows0, rows1,
               si0, si1, si2, si3, sg0, sg1):
    cid = lax.axis_index("c")
    sid = lax.axis_index("s")
    base, cnt = _chunk_base(cid, sid)
    ibuf = (i0, i1, i2, i3)
    isem = (si0, si1, si2, si3)
    rows = (rows0, rows1)
    gsem = (sg0, sg1)
    pltpu.sync_copy(zacc_hbm.at[pl.ds(sid * RPT, RPT)],
                    acc_sh.at[pl.ds(sid * RPT, RPT)])
    plsc.subcore_barrier()

    pltpu.async_copy(idx_hbm.at[base], ibuf[0], isem[0])
    pltpu.async_copy(idx_hbm.at[base + 1], ibuf[1], isem[1])
    pltpu.make_async_copy(idx_hbm.at[base], ibuf[0], isem[0]).wait()
    pltpu.async_copy(g_hbm.at[ibuf[0].at[0]], rows[0], gsem[0])

    def outer(c4, carry):
        for u in range(4):
            c = c4 * 4 + u
            b2 = u % 2
            bi = u

            @pl.when(c + 2 < cnt)
            def _():
                pltpu.async_copy(idx_hbm.at[base + c + 2],
                                 ibuf[(bi + 2) % 4], isem[(bi + 2) % 4])

            @pl.when(c + 1 < cnt)
            def _():
                pltpu.make_async_copy(idx_hbm.at[base],
                                      ibuf[(bi + 1) % 4],
                                      isem[(bi + 1) % 4]).wait()
                pltpu.async_copy(g_hbm.at[ibuf[(bi + 1) % 4].at[0]],
                                 rows[1 - b2], gsem[1 - b2])

            @pl.when(c < cnt)
            def _():
                pltpu.make_async_copy(g_hbm.at[pl.ds(0, K)],
                                      rows[b2], gsem[b2]).wait()
                pltpu.sync_copy(rows[b2], acc_sh.at[ibuf[bi].at[1]], add=True)
        return carry

    lax.fori_loop(0, CH_MAX // 4, outer, 0)
    plsc.subcore_barrier()
    pltpu.sync_copy(acc_sh.at[pl.ds(sid * RPT, RPT)],
                    out_hbm.at[cid, pl.ds(sid * RPT, RPT)])


_scat_pass = functools.partial(
    pl.kernel,
    out_type=jax.ShapeDtypeStruct((NC, NACC, D), jnp.float32),
    mesh=_sc_mesh,
    scratch_types=[
        pltpu.VMEM_SHARED((NACC, D), jnp.float32),
        pltpu.VMEM((2, K), jnp.int32),
        pltpu.VMEM((2, K), jnp.int32),
        pltpu.VMEM((2, K), jnp.int32),
        pltpu.VMEM((2, K), jnp.int32),
        pltpu.VMEM((K, D), jnp.float32),
        pltpu.VMEM((K, D), jnp.float32),
        pltpu.SemaphoreType.DMA,
        pltpu.SemaphoreType.DMA,
        pltpu.SemaphoreType.DMA,
        pltpu.SemaphoreType.DMA,
        pltpu.SemaphoreType.DMA,
        pltpu.SemaphoreType.DMA,
    ],
)(_scat_body)


BR = 1264
_TC_GRID = NACC // BR


def _g1_body(x_ref, degp_ref, w_ref, g_ref, dinv_ref):
    deg = jnp.sum(degp_ref[0] + degp_ref[1], axis=-1) * (1.0 / D) + 1.0
    dinv = (1.0 / jnp.sqrt(deg))[:, None]
    dinv_ref[:, :] = jnp.broadcast_to(dinv, (BR, D))
    g_ref[:, :] = jnp.dot(x_ref[:, :], w_ref[:, :],
                          preferred_element_type=jnp.float32) * dinv


def _mid_body(acc_ref, g_ref, dinv_ref, b_ref, w_ref, out_ref):
    dinv = dinv_ref[:, :]
    v = (acc_ref[0] + acc_ref[1] + g_ref[:, :]) * dinv + b_ref[:]
    xn = jnp.maximum(v, 0.01 * v)
    out_ref[:, :] = jnp.dot(xn, w_ref[:, :],
                            preferred_element_type=jnp.float32) * dinv


def _final_body(acc_ref, g_ref, dinv_ref, b_ref, wl_ref, bl_ref, out_ref):
    dinv = dinv_ref[:, :]
    v = (acc_ref[0] + acc_ref[1] + g_ref[:, :]) * dinv + b_ref[:]
    xn = jnp.maximum(v, 0.01 * v)
    out_ref[:, :] = jnp.dot(xn, wl_ref[:, :],
                            preferred_element_type=jnp.float32) + bl_ref[:]


_row_spec = pl.BlockSpec((BR, D), lambda i: (i, 0))
_acc_spec = pl.BlockSpec((NC, BR, D), lambda i: (0, i, 0))
_w_spec = pl.BlockSpec((D, D), lambda i: (0, 0))
_b_spec = pl.BlockSpec((D,), lambda i: (0,))


def _tc_g1(x_pad, degp, w1):
    return pl.pallas_call(
        _g1_body,
        out_shape=[jax.ShapeDtypeStruct((NACC, D), jnp.float32),
                   jax.ShapeDtypeStruct((NACC, D), jnp.float32)],
        grid=(_TC_GRID,),
        in_specs=[_row_spec, _acc_spec, _w_spec],
        out_specs=[_row_spec, _row_spec],
    )(x_pad, degp, w1)


def _tc_mid(acc, g, dinv, b, w):
    return pl.pallas_call(
        _mid_body,
        out_shape=jax.ShapeDtypeStruct((NACC, D), jnp.float32),
        grid=(_TC_GRID,),
        in_specs=[_acc_spec, _row_spec, _row_spec, _b_spec, _w_spec],
        out_specs=_row_spec,
    )(acc, g, dinv, b, w)


def _tc_final(acc, g, dinv, b, wl, bl):
    return pl.pallas_call(
        _final_body,
        out_shape=jax.ShapeDtypeStruct((NACC, C), jnp.float32),
        grid=(_TC_GRID,),
        in_specs=[_acc_spec, _row_spec, _row_spec, _b_spec,
                  pl.BlockSpec((D, C), lambda i: (0, 0)),
                  pl.BlockSpec((C,), lambda i: (0,))],
        out_specs=pl.BlockSpec((BR, C), lambda i: (i, 0)),
    )(acc, g, dinv, b, wl, bl)


def kernel(x, edge_index, W1, b1, W2, b2, W3, b3, Wl, bl):
    ei = edge_index.astype(jnp.int32)
    pad = jnp.full((IDX_ROWS * K - E,), N, dtype=jnp.int32)
    src = jnp.concatenate([ei[0], pad]).reshape(IDX_ROWS, K)
    dst = jnp.concatenate([ei[1], pad]).reshape(IDX_ROWS, K)
    idx2 = jnp.stack([src, dst], axis=1)
    x_pad = jnp.pad(x, ((0, NACC - N), (0, 0)))

    ones_rows = jnp.ones((K, D), jnp.float32)
    zacc = jnp.zeros((NACC, D), jnp.float32)

    degp = _deg_pass(dst, ones_rows, zacc)

    g1, dinv = _tc_g1(x_pad, degp, W1)
    acc1 = _scat_pass(g1, idx2, zacc)
    g2 = _tc_mid(acc1, g1, dinv, b1, W2)
    acc2 = _scat_pass(g2, idx2, zacc)
    g3 = _tc_mid(acc2, g2, dinv, b2, W3)
    acc3 = _scat_pass(g3, idx2, zacc)
    out = _tc_final(acc3, g3, dinv, b3, Wl, bl)
    return out[:N]
